# Initial kernel scaffold; baseline (speedup 1.0000x reference)
import jax
import jax.numpy as jnp
from jax.experimental import pallas as pl


def kernel(num_prop, cat_prop, w_gate, W1, b1, W2, b2, W_out, b_out):
    raise NotImplementedError("write your pallas kernel here")


# speedup vs baseline: 2.0672x; 2.0672x over previous
import jax
import jax.numpy as jnp
from jax.experimental import pallas as pl
from jax.experimental.pallas import tpu as pltpu

_E = 8
_K = 2
_IN = 9
_HID = 128
_MOE_OUT = 32
_OUT = 2
_B = 16384
_LOSS_COF = 0.01
_BLK = 2048


def _cv(v2d):
    mean = jnp.sum(v2d) / _E
    var = jnp.sum((v2d - mean) ** 2) / (_E - 1)
    return var / (mean * mean + 1e-10)


def _moe_body(x_ref, wg_ref, w1_ref, b1_ref, w2_ref, b2_ref, wo_ref, bo_ref,
              out_ref, loss_ref, acc_ref):
    i = pl.program_id(0)
    x = x_ref[:]

    logits = jnp.dot(x, wg_ref[:], preferred_element_type=jnp.float32)
    iota = jax.lax.broadcasted_iota(jnp.int32, logits.shape, 1)
    i1 = jnp.argmax(logits, axis=1)[:, None]
    oh1 = iota == i1
    l1 = jnp.max(logits, axis=1, keepdims=True)
    masked = jnp.where(oh1, -jnp.inf, logits)
    i2 = jnp.argmax(masked, axis=1)[:, None]
    oh2 = iota == i2
    l2 = jnp.max(masked, axis=1, keepdims=True)
    e2 = jnp.exp(l2 - l1)
    denom = 1.0 + e2
    g1 = 1.0 / denom
    g2 = e2 / denom
    gates = jnp.where(oh1, g1, 0.0) + jnp.where(oh2, g2, 0.0)

    h = jnp.dot(x, w1_ref[:], preferred_element_type=jnp.float32) + b1_ref[:]
    h = jnp.maximum(h, 0.0)

    y = jnp.zeros((x.shape[0], _MOE_OUT), dtype=jnp.float32)
    for e in range(_E):
        he = h[:, e * _HID:(e + 1) * _HID]
        oe = jnp.dot(he, w2_ref[e], preferred_element_type=jnp.float32) + b2_ref[e]
        oe = oe - jnp.max(oe, axis=1, keepdims=True)
        pe = jnp.exp(oe)
        pe = pe / jnp.sum(pe, axis=1, keepdims=True)
        y = y + gates[:, e:e + 1] * pe

    out_ref[:] = jnp.dot(y, wo_ref[:], preferred_element_type=jnp.float32) + bo_ref[:]

    imp_part = jnp.sum(gates, axis=0, keepdims=True)
    load_part = jnp.sum((gates > 0.0).astype(jnp.float32), axis=0, keepdims=True)

    @pl.when(i == 0)
    def _():
        acc_ref[:] = jnp.zeros_like(acc_ref)

    acc_ref[0:1, :] = acc_ref[0:1, :] + imp_part
    acc_ref[1:2, :] = acc_ref[1:2, :] + load_part

    @pl.when(i == pl.num_programs(0) - 1)
    def _():
        val = (_cv(acc_ref[0:1, :]) + _cv(acc_ref[1:2, :])) * _LOSS_COF
        loss_ref[:, :] = jnp.reshape(val, (1, 1))


def kernel(num_prop, cat_prop, w_gate, W1, b1, W2, b2, W_out, b_out):
    del cat_prop
    w1_all = jnp.transpose(W1, (1, 0, 2)).reshape(_IN, _E * _HID)
    b1_all = b1.reshape(1, _E * _HID)
    b2_r = b2.reshape(_E, 1, _MOE_OUT)
    bo_r = b_out.reshape(1, _OUT)

    grid = _B // _BLK
    rep = lambda i: (0, 0)
    out, loss = pl.pallas_call(
        _moe_body,
        grid=(grid,),
        in_specs=[
            pl.BlockSpec((_BLK, _IN), lambda i: (i, 0)),
            pl.BlockSpec((_IN, _E), rep),
            pl.BlockSpec((_IN, _E * _HID), rep),
            pl.BlockSpec((1, _E * _HID), rep),
            pl.BlockSpec((_E, _HID, _MOE_OUT), lambda i: (0, 0, 0)),
            pl.BlockSpec((_E, 1, _MOE_OUT), lambda i: (0, 0, 0)),
            pl.BlockSpec((_MOE_OUT, _OUT), rep),
            pl.BlockSpec((1, _OUT), rep),
        ],
        out_specs=[
            pl.BlockSpec((_BLK, _OUT), lambda i: (i, 0)),
            pl.BlockSpec((1, 1), rep),
        ],
        out_shape=[
            jax.ShapeDtypeStruct((_B, _OUT), jnp.float32),
            jax.ShapeDtypeStruct((1, 1), jnp.float32),
        ],
        scratch_shapes=[pltpu.VMEM((2, _E), jnp.float32)],
    )(num_prop, w_gate, w1_all, b1_all, W2, b2_r, W_out, bo_r)
    return out, loss[0, 0]


# speedup vs baseline: 3.1277x; 1.5130x over previous
import jax
import jax.numpy as jnp
from jax.experimental import pallas as pl
from jax.experimental.pallas import tpu as pltpu

_E = 8
_K = 2
_IN = 9
_HID = 128
_MOE_OUT = 32
_OUT = 2
_B = 16384
_LOSS_COF = 0.01
_BLK = 2048


def _cv(v2d):
    mean = jnp.sum(v2d) / _E
    var = jnp.sum((v2d - mean) ** 2) / (_E - 1)
    return var / (mean * mean + 1e-10)


def _moe_body(x_ref, wg_ref, w1_ref, b1_ref, w2_ref, b2_ref, gsum_ref,
              gbc_ref, wot_ref, bo_ref, out_ref, loss_ref, acc_ref):
    i = pl.program_id(0)
    x = x_ref[:]

    logits = jnp.dot(x, wg_ref[:], preferred_element_type=jnp.float32)
    iota = jax.lax.broadcasted_iota(jnp.int32, logits.shape, 1)
    i1 = jnp.argmax(logits, axis=1)[:, None]
    oh1 = iota == i1
    l1 = jnp.max(logits, axis=1, keepdims=True)
    masked = jnp.where(oh1, -jnp.inf, logits)
    i2 = jnp.argmax(masked, axis=1)[:, None]
    oh2 = iota == i2
    l2 = jnp.max(masked, axis=1, keepdims=True)
    e2 = jnp.exp(l2 - l1)
    denom = 1.0 + e2
    g1 = 1.0 / denom
    g2 = e2 / denom
    gates = jnp.where(oh1, g1, 0.0) + jnp.where(oh2, g2, 0.0)

    h = jnp.dot(x, w1_ref[:], preferred_element_type=jnp.float32) + b1_ref[:]
    h = jnp.maximum(h, 0.0)

    o_all = jnp.dot(h, w2_ref[:], preferred_element_type=jnp.float32) + b2_ref[:]
    o_all = o_all - jnp.max(o_all, axis=1, keepdims=True)
    ex = jnp.exp(o_all)
    s = jnp.dot(ex, gsum_ref[:], preferred_element_type=jnp.float32)
    w = jnp.where(gates > 0.0, gates / s, 0.0)
    wbc = jnp.dot(w, gbc_ref[:], preferred_element_type=jnp.float32)
    weighted = ex * wbc
    out_ref[:] = jnp.dot(weighted, wot_ref[:],
                         preferred_element_type=jnp.float32) + bo_ref[:]

    imp_part = jnp.sum(gates, axis=0, keepdims=True)
    load_part = jnp.sum((gates > 0.0).astype(jnp.float32), axis=0, keepdims=True)

    @pl.when(i == 0)
    def _():
        acc_ref[:] = jnp.zeros_like(acc_ref)

    acc_ref[0:1, :] = acc_ref[0:1, :] + imp_part
    acc_ref[1:2, :] = acc_ref[1:2, :] + load_part

    @pl.when(i == pl.num_programs(0) - 1)
    def _():
        val = (_cv(acc_ref[0:1, :]) + _cv(acc_ref[1:2, :])) * _LOSS_COF
        loss_ref[:, :] = jnp.reshape(val, (1, 1))


def kernel(num_prop, cat_prop, w_gate, W1, b1, W2, b2, W_out, b_out):
    del cat_prop
    f32 = jnp.float32
    w1_all = jnp.transpose(W1, (1, 0, 2)).reshape(_IN, _E * _HID)
    b1_all = b1.reshape(1, _E * _HID)
    eye_e = jnp.eye(_E, dtype=f32)
    w2_bd = (eye_e[:, None, :, None] * jnp.transpose(W2, (0, 1, 2))[:, :, None, :]
             ).reshape(_E * _HID, _E * _MOE_OUT)
    b2_all = b2.reshape(1, _E * _MOE_OUT)
    gsum = jnp.repeat(eye_e, _MOE_OUT, axis=0)
    gbc = gsum.T
    wo_t = jnp.tile(W_out, (_E, 1))
    bo_r = b_out.reshape(1, _OUT)

    grid = _B // _BLK
    rep = lambda i: (0, 0)
    out, loss = pl.pallas_call(
        _moe_body,
        grid=(grid,),
        in_specs=[
            pl.BlockSpec((_BLK, _IN), lambda i: (i, 0)),
            pl.BlockSpec((_IN, _E), rep),
            pl.BlockSpec((_IN, _E * _HID), rep),
            pl.BlockSpec((1, _E * _HID), rep),
            pl.BlockSpec((_E * _HID, _E * _MOE_OUT), rep),
            pl.BlockSpec((1, _E * _MOE_OUT), rep),
            pl.BlockSpec((_E * _MOE_OUT, _E), rep),
            pl.BlockSpec((_E, _E * _MOE_OUT), rep),
            pl.BlockSpec((_E * _MOE_OUT, _OUT), rep),
            pl.BlockSpec((1, _OUT), rep),
        ],
        out_specs=[
            pl.BlockSpec((_BLK, _OUT), lambda i: (i, 0)),
            pl.BlockSpec((1, 1), rep),
        ],
        out_shape=[
            jax.ShapeDtypeStruct((_B, _OUT), jnp.float32),
            jax.ShapeDtypeStruct((1, 1), jnp.float32),
        ],
        scratch_shapes=[pltpu.VMEM((2, _E), jnp.float32)],
    )(num_prop, w_gate, w1_all, b1_all, w2_bd, b2_all, gsum, gbc, wo_t, bo_r)
    return out, loss[0, 0]


# speedup vs baseline: 3.1605x; 1.0105x over previous
import jax
import jax.numpy as jnp
from jax.experimental import pallas as pl
from jax.experimental.pallas import tpu as pltpu

_E = 8
_K = 2
_IN = 9
_HID = 128
_MOE_OUT = 32
_OUT = 2
_B = 16384
_LOSS_COF = 0.01
_BLK = 4096


def _cv(v2d):
    mean = jnp.sum(v2d) / _E
    var = jnp.sum((v2d - mean) ** 2) / (_E - 1)
    return var / (mean * mean + 1e-10)


def _moe_body(x_ref, wg_ref, w1_ref, b1_ref, w2_ref, b2_ref, gsum_ref,
              gbc_ref, wot_ref, bo_ref, out_ref, loss_ref, acc_ref):
    i = pl.program_id(0)
    x = x_ref[:]

    logits = jnp.dot(x, wg_ref[:], preferred_element_type=jnp.float32)
    iota = jax.lax.broadcasted_iota(jnp.int32, logits.shape, 1)
    i1 = jnp.argmax(logits, axis=1)[:, None]
    oh1 = iota == i1
    l1 = jnp.max(logits, axis=1, keepdims=True)
    masked = jnp.where(oh1, -jnp.inf, logits)
    i2 = jnp.argmax(masked, axis=1)[:, None]
    oh2 = iota == i2
    l2 = jnp.max(masked, axis=1, keepdims=True)
    e2 = jnp.exp(l2 - l1)
    denom = 1.0 + e2
    g1 = 1.0 / denom
    g2 = e2 / denom
    gates = jnp.where(oh1, g1, 0.0) + jnp.where(oh2, g2, 0.0)

    h = jnp.dot(x, w1_ref[:], preferred_element_type=jnp.float32) + b1_ref[:]
    h = jnp.maximum(h, 0.0)

    o_all = jnp.dot(h, w2_ref[:], preferred_element_type=jnp.float32) + b2_ref[:]
    o_all = o_all - jnp.max(o_all, axis=1, keepdims=True)
    ex = jnp.exp(o_all)
    s = jnp.dot(ex, gsum_ref[:], preferred_element_type=jnp.float32)
    w = jnp.where(gates > 0.0, gates / s, 0.0)
    wbc = jnp.dot(w, gbc_ref[:], preferred_element_type=jnp.float32)
    weighted = ex * wbc
    out_ref[:] = jnp.dot(weighted, wot_ref[:],
                         preferred_element_type=jnp.float32) + bo_ref[:]

    imp_part = jnp.sum(gates, axis=0, keepdims=True)
    load_part = jnp.sum((gates > 0.0).astype(jnp.float32), axis=0, keepdims=True)

    @pl.when(i == 0)
    def _():
        acc_ref[:] = jnp.zeros_like(acc_ref)

    acc_ref[0:1, :] = acc_ref[0:1, :] + imp_part
    acc_ref[1:2, :] = acc_ref[1:2, :] + load_part

    @pl.when(i == pl.num_programs(0) - 1)
    def _():
        val = (_cv(acc_ref[0:1, :]) + _cv(acc_ref[1:2, :])) * _LOSS_COF
        loss_ref[:, :] = jnp.reshape(val, (1, 1))


def kernel(num_prop, cat_prop, w_gate, W1, b1, W2, b2, W_out, b_out):
    del cat_prop
    f32 = jnp.float32
    w1_all = jnp.transpose(W1, (1, 0, 2)).reshape(_IN, _E * _HID)
    b1_all = b1.reshape(1, _E * _HID)
    eye_e = jnp.eye(_E, dtype=f32)
    w2_bd = (eye_e[:, None, :, None] * jnp.transpose(W2, (0, 1, 2))[:, :, None, :]
             ).reshape(_E * _HID, _E * _MOE_OUT)
    b2_all = b2.reshape(1, _E * _MOE_OUT)
    gsum = jnp.repeat(eye_e, _MOE_OUT, axis=0)
    gbc = gsum.T
    wo_t = jnp.tile(W_out, (_E, 1))
    bo_r = b_out.reshape(1, _OUT)

    grid = _B // _BLK
    rep = lambda i: (0, 0)
    out, loss = pl.pallas_call(
        _moe_body,
        grid=(grid,),
        in_specs=[
            pl.BlockSpec((_BLK, _IN), lambda i: (i, 0)),
            pl.BlockSpec((_IN, _E), rep),
            pl.BlockSpec((_IN, _E * _HID), rep),
            pl.BlockSpec((1, _E * _HID), rep),
            pl.BlockSpec((_E * _HID, _E * _MOE_OUT), rep),
            pl.BlockSpec((1, _E * _MOE_OUT), rep),
            pl.BlockSpec((_E * _MOE_OUT, _E), rep),
            pl.BlockSpec((_E, _E * _MOE_OUT), rep),
            pl.BlockSpec((_E * _MOE_OUT, _OUT), rep),
            pl.BlockSpec((1, _OUT), rep),
        ],
        out_specs=[
            pl.BlockSpec((_BLK, _OUT), lambda i: (i, 0)),
            pl.BlockSpec((1, 1), rep),
        ],
        out_shape=[
            jax.ShapeDtypeStruct((_B, _OUT), jnp.float32),
            jax.ShapeDtypeStruct((1, 1), jnp.float32),
        ],
        scratch_shapes=[pltpu.VMEM((2, _E), jnp.float32)],
    )(num_prop, w_gate, w1_all, b1_all, w2_bd, b2_all, gsum, gbc, wo_t, bo_r)
    return out, loss[0, 0]
